# speedup vs baseline: 1.0855x; 1.0855x over previous
import functools

import jax
import jax.numpy as jnp
from jax import lax
from jax.experimental import pallas as pl
from jax.experimental.pallas import tpu as pltpu
from jax.experimental.pallas import tpu_sc as plsc

_N_BINS = 51
_D = 768


@functools.partial(jax.jit, static_argnames=("n_tokens",))
def _binned_lookup(flat_values, table, *, n_tokens):
    info = plsc.get_sparse_core_info()
    nc, ns, lanes = info.num_cores, info.num_subcores, info.num_lanes
    nw = nc * ns
    bpw = n_tokens // nw
    chunk = 128
    n_chunks = bpw // chunk

    mesh = plsc.VectorSubcoreMesh(core_axis_name="c", subcore_axis_name="s")

    @functools.partial(
        pl.kernel,
        mesh=mesh,
        out_type=jax.ShapeDtypeStruct((n_tokens, _D), jnp.float32),
        scratch_types=[
            pltpu.VMEM((bpw,), jnp.float32),
            pltpu.VMEM((bpw,), jnp.int32),
            pltpu.VMEM((chunk, _D), jnp.float32),
            pltpu.SemaphoreType.DMA,
        ],
    )
    def sc_kernel(vals_hbm, table_hbm, out_hbm, vals_v, idx_v, rows_v, sem):
        wid = lax.axis_index("s") * nc + lax.axis_index("c")
        base = wid * bpw
        pltpu.sync_copy(vals_hbm.at[pl.ds(base, bpw)], vals_v)

        def cvt(i, carry):
            v = vals_v[pl.ds(i * lanes, lanes)]
            b = (v * (_N_BINS - 1)).astype(jnp.int32)
            idx_v[pl.ds(i * lanes, lanes)] = jnp.clip(b, 0, _N_BINS - 1)
            return carry

        lax.fori_loop(0, bpw // lanes, cvt, 0, unroll=4)

        def do_chunk(c, carry):
            pltpu.async_copy(
                table_hbm.at[idx_v.at[pl.ds(c * chunk, chunk)]], rows_v, sem
            ).wait()
            pltpu.sync_copy(rows_v, out_hbm.at[pl.ds(base + c * chunk, chunk)])
            return carry

        lax.fori_loop(0, n_chunks, do_chunk, 0)

    return sc_kernel(flat_values, table)


def kernel(values, embedding_weight):
    batch, seq = values.shape
    flat = values.reshape(batch * seq)
    out = _binned_lookup(flat, embedding_weight, n_tokens=batch * seq)
    return out.reshape(batch, seq, _D)
